# baseline (speedup 1.0000x reference)
import jax
import jax.numpy as jnp
from jax.experimental import pallas as pl
from jax.experimental.pallas import tpu as pltpu

D = 768
H = 12
HD = 64
E = 64
K = 8
SBLK = 2048
MBLK = 1024
NMB = SBLK // MBLK


def _dot(a, b, dims):
    return jax.lax.dot_general(a, b, (dims, ((), ())),
                               preferred_element_type=jnp.float32)


def _gate_kernel(t_ref, mt_ref, g_ref, be_ref, wq_ref, bq_ref, wk_ref, bk_ref,
                 wv_ref, bv_ref, wo_ref, bo_ref, gw_ref, h_ref,
                 idx_ref, w_ref, aux_ref,
                 A_s, sc_s, mx_s, den_s, ctx_s, lg_s):
    b = pl.program_id(0)
    p = pl.program_id(1)
    s = pl.program_id(2)
    nb = pl.num_programs(0)
    ns = pl.num_programs(2)

    headmask = (jax.lax.broadcasted_iota(jnp.int32, (H, D), 1) // HD
                == jax.lax.broadcasted_iota(jnp.int32, (H, D), 0))

    @pl.when((b == 0) & (p == 0) & (s == 0))
    def _prologue():
        x = mt_ref[...]
        mu = jnp.mean(x, axis=1, keepdims=True)
        var = jnp.mean((x - mu) ** 2, axis=1, keepdims=True)
        qin = (x - mu) / jnp.sqrt(var + 1e-6) * g_ref[...] + be_ref[...]
        q = _dot(qin, wq_ref[...], ((1,), (1,))) + bq_ref[...]
        A_s[...] = jnp.where(headmask, jnp.broadcast_to(q, (H, D)), 0.0)

    @pl.when(p == 0)
    def _scores():
        for i in range(NMB):
            kb = _dot(h_ref[0, i * MBLK:(i + 1) * MBLK], wk_ref[...],
                      ((1,), (1,))) + bk_ref[...]
            sc_s[:, pl.ds(s * SBLK + i * MBLK, MBLK)] = (
                _dot(A_s[...], kb, ((1,), (1,))) * 0.125)

    @pl.when((p == 1) & (s == 0))
    def _softmax_stats():
        sc = sc_s[...]
        mx = jnp.max(sc, axis=1, keepdims=True)
        den = jnp.sum(jnp.exp(sc - mx), axis=1, keepdims=True)
        mx_s[...] = mx
        den_s[...] = den
        ctx_s[...] = jnp.zeros((H, D), jnp.float32)

    @pl.when(p == 1)
    def _pool():
        for i in range(NMB):
            vb = _dot(h_ref[0, i * MBLK:(i + 1) * MBLK], wv_ref[...],
                      ((1,), (1,))) + bv_ref[...]
            att = (jnp.exp(sc_s[:, pl.ds(s * SBLK + i * MBLK, MBLK)]
                           - mx_s[...]) / den_s[...])
            ctx_s[...] += _dot(att, vb, ((1,), (0,)))

    @pl.when((p == 1) & (s == ns - 1))
    def _per_batch():
        ctx = jnp.sum(jnp.where(headmask, ctx_s[...], 0.0), axis=0,
                      keepdims=True)
        out = _dot(ctx, wo_ref[...], ((1,), (1,))) + bo_ref[...]
        lg_s[pl.ds(b, 1), :] = _dot(out, gw_ref[...], ((1,), (1,)))

    @pl.when((b == nb - 1) & (p == 1) & (s == ns - 1))
    def _routing():
        lg = lg_s[...]
        nbat = lg.shape[0]
        rowmax = jnp.max(lg, axis=1, keepdims=True)
        ex = jnp.exp(lg - rowmax)
        se = jnp.sum(ex, axis=1, keepdims=True)
        colio = jax.lax.broadcasted_iota(jnp.int32, (nbat, E), 1)
        work = ex / se
        idx_cols, w_cols = [], []
        tot = jnp.zeros((nbat, 1), jnp.float32)
        for _ in range(K):
            mx = jnp.max(work, axis=1, keepdims=True)
            cand = jnp.where(work == mx, colio, E)
            ix = jnp.min(cand, axis=1, keepdims=True)
            idx_cols.append(ix)
            w_cols.append(mx)
            tot = tot + mx
            work = jnp.where(colio == ix, -1.0, work)
        idx_ref[...] = jnp.concatenate(idx_cols, axis=1)
        w_ref[...] = jnp.concatenate(w_cols, axis=1) / (tot + 1e-20)
        t = t_ref[0]
        lse = jnp.log(se) + rowmax
        lt = jnp.sum(jnp.where(colio == t, lg, 0.0), axis=1, keepdims=True)
        aux_ref[...] = jnp.mean(lse - lt, keepdims=True)


def kernel(hidden_states, ln_gamma, ln_beta, moe_tokens, Wq, bq, Wk, bk,
           Wv, bv, Wo, bo, gate_weight, target_expert):
    B, S, d = hidden_states.shape
    ns = S // SBLK
    row = lambda v: v.reshape(1, d)
    t = jnp.asarray(target_expert, jnp.int32).reshape(1)

    const = lambda shape: pl.BlockSpec(shape, lambda b, p, s: (0,) * len(shape))

    idx, w, aux = pl.pallas_call(
        _gate_kernel,
        grid=(B, 2, ns),
        in_specs=[
            pl.BlockSpec(memory_space=pltpu.SMEM),
            const((1, d)),
            const((1, d)), const((1, d)),
            const((d, d)), const((1, d)),
            const((d, d)), const((1, d)),
            const((d, d)), const((1, d)),
            const((d, d)), const((1, d)),
            const((E, d)),
            pl.BlockSpec((1, SBLK, d), lambda b, p, s: (b, s, 0)),
        ],
        out_specs=[const((B, K)), const((B, K)), const((1, 1))],
        out_shape=[
            jax.ShapeDtypeStruct((B, K), jnp.int32),
            jax.ShapeDtypeStruct((B, K), jnp.float32),
            jax.ShapeDtypeStruct((1, 1), jnp.float32),
        ],
        scratch_shapes=[
            pltpu.VMEM((H, D), jnp.float32),
            pltpu.VMEM((H, S), jnp.float32),
            pltpu.VMEM((H, 1), jnp.float32),
            pltpu.VMEM((H, 1), jnp.float32),
            pltpu.VMEM((H, D), jnp.float32),
            pltpu.VMEM((B, E), jnp.float32),
        ],
        compiler_params=pltpu.CompilerParams(
            dimension_semantics=("arbitrary", "arbitrary", "arbitrary")),
    )(t, moe_tokens.reshape(1, d), row(ln_gamma), row(ln_beta),
      Wq, row(bq), Wk, row(bk), Wv, row(bv), Wo, row(bo), gate_weight,
      hidden_states)
    return idx, w, aux.reshape(())


# speedup vs baseline: 1.1570x; 1.1570x over previous
import jax
import jax.numpy as jnp
from jax.experimental import pallas as pl
from jax.experimental.pallas import tpu as pltpu

D = 768
H = 12
HD = 64
E = 64
K = 8
SBLK = 2048
MBLK = 1024
NMB = SBLK // MBLK


def _dot(a, b, dims):
    return jax.lax.dot_general(a, b, (dims, ((), ())),
                               preferred_element_type=jnp.float32)


def _gate_kernel(t_ref, mt_ref, g_ref, be_ref, wq_ref, bq_ref, wk_ref, bk_ref,
                 wv_ref, bv_ref, wo_ref, bo_ref, gw_ref, h_ref,
                 idx_ref, w_ref, aux_ref,
                 A_s, sc_s, v_s, lg_s):
    b = pl.program_id(0)
    s = pl.program_id(1)
    nb = pl.num_programs(0)
    ns = pl.num_programs(1)

    headmask = (jax.lax.broadcasted_iota(jnp.int32, (H, D), 1) // HD
                == jax.lax.broadcasted_iota(jnp.int32, (H, D), 0))

    @pl.when((b == 0) & (s == 0))
    def _prologue():
        x = mt_ref[...]
        mu = jnp.mean(x, axis=1, keepdims=True)
        var = jnp.mean((x - mu) ** 2, axis=1, keepdims=True)
        qin = (x - mu) / jnp.sqrt(var + 1e-6) * g_ref[...] + be_ref[...]
        q = _dot(qin, wq_ref[...], ((1,), (1,))) + bq_ref[...]
        A_s[...] = jnp.where(headmask, jnp.broadcast_to(q, (H, D)), 0.0)

    for i in range(NMB):
        hb = h_ref[0, i * MBLK:(i + 1) * MBLK]
        kb = _dot(hb, wk_ref[...], ((1,), (1,))) + bk_ref[...]
        sc_s[:, pl.ds(s * SBLK + i * MBLK, MBLK)] = (
            _dot(A_s[...], kb, ((1,), (1,))) * 0.125)
        v_s[pl.ds(s * SBLK + i * MBLK, MBLK), :] = (
            _dot(hb, wv_ref[...], ((1,), (1,))) + bv_ref[...])

    @pl.when(s == ns - 1)
    def _per_batch():
        sc = sc_s[...]
        mx = jnp.max(sc, axis=1, keepdims=True)
        den = jnp.sum(jnp.exp(sc - mx), axis=1, keepdims=True)
        acc = jnp.zeros((H, D), jnp.float32)
        nch = (ns * SBLK) // MBLK
        for j in range(nch):
            att = jnp.exp(sc_s[:, j * MBLK:(j + 1) * MBLK] - mx) / den
            acc = acc + _dot(att, v_s[j * MBLK:(j + 1) * MBLK, :],
                             ((1,), (0,)))
        ctx = jnp.sum(jnp.where(headmask, acc, 0.0), axis=0,
                      keepdims=True)
        out = _dot(ctx, wo_ref[...], ((1,), (1,))) + bo_ref[...]
        lg_s[pl.ds(b, 1), :] = _dot(out, gw_ref[...], ((1,), (1,)))

    @pl.when((b == nb - 1) & (s == ns - 1))
    def _routing():
        lg = lg_s[...]
        nbat = lg.shape[0]
        rowmax = jnp.max(lg, axis=1, keepdims=True)
        ex = jnp.exp(lg - rowmax)
        se = jnp.sum(ex, axis=1, keepdims=True)
        colio = jax.lax.broadcasted_iota(jnp.int32, (nbat, E), 1)
        work = ex / se
        idx_cols, w_cols = [], []
        tot = jnp.zeros((nbat, 1), jnp.float32)
        for _ in range(K):
            mx = jnp.max(work, axis=1, keepdims=True)
            cand = jnp.where(work == mx, colio, E)
            ix = jnp.min(cand, axis=1, keepdims=True)
            idx_cols.append(ix)
            w_cols.append(mx)
            tot = tot + mx
            work = jnp.where(colio == ix, -1.0, work)
        idx_ref[...] = jnp.concatenate(idx_cols, axis=1)
        w_ref[...] = jnp.concatenate(w_cols, axis=1) / (tot + 1e-20)
        t = t_ref[0]
        lse = jnp.log(se) + rowmax
        lt = jnp.sum(jnp.where(colio == t, lg, 0.0), axis=1, keepdims=True)
        aux_ref[...] = jnp.mean(lse - lt, keepdims=True)


def kernel(hidden_states, ln_gamma, ln_beta, moe_tokens, Wq, bq, Wk, bk,
           Wv, bv, Wo, bo, gate_weight, target_expert):
    B, S, d = hidden_states.shape
    ns = S // SBLK
    row = lambda v: v.reshape(1, d)
    t = jnp.asarray(target_expert, jnp.int32).reshape(1)

    const = lambda shape: pl.BlockSpec(shape, lambda b, s: (0,) * len(shape))

    idx, w, aux = pl.pallas_call(
        _gate_kernel,
        grid=(B, ns),
        in_specs=[
            pl.BlockSpec(memory_space=pltpu.SMEM),
            const((1, d)),
            const((1, d)), const((1, d)),
            const((d, d)), const((1, d)),
            const((d, d)), const((1, d)),
            const((d, d)), const((1, d)),
            const((d, d)), const((1, d)),
            const((E, d)),
            pl.BlockSpec((1, SBLK, d), lambda b, s: (b, s, 0)),
        ],
        out_specs=[const((B, K)), const((B, K)), const((1, 1))],
        out_shape=[
            jax.ShapeDtypeStruct((B, K), jnp.int32),
            jax.ShapeDtypeStruct((B, K), jnp.float32),
            jax.ShapeDtypeStruct((1, 1), jnp.float32),
        ],
        scratch_shapes=[
            pltpu.VMEM((H, D), jnp.float32),
            pltpu.VMEM((H, S), jnp.float32),
            pltpu.VMEM((S, D), jnp.float32),
            pltpu.VMEM((B, E), jnp.float32),
        ],
        compiler_params=pltpu.CompilerParams(
            dimension_semantics=("arbitrary", "arbitrary")),
    )(t, moe_tokens.reshape(1, d), row(ln_gamma), row(ln_beta),
      Wq, row(bq), Wk, row(bk), Wv, row(bv), Wo, row(bo), gate_weight,
      hidden_states)
    return idx, w, aux.reshape(())


# speedup vs baseline: 1.2126x; 1.0481x over previous
import jax
import jax.numpy as jnp
from jax.experimental import pallas as pl
from jax.experimental.pallas import tpu as pltpu

D = 768
H = 12
HD = 64
E = 64
K = 8
SBLK = 4096
MBLK = 1024
NMB = SBLK // MBLK


def _dot(a, b, dims):
    return jax.lax.dot_general(a, b, (dims, ((), ())),
                               preferred_element_type=jnp.float32)


def _headmask():
    return (jax.lax.broadcasted_iota(jnp.int32, (H, D), 1) // HD
            == jax.lax.broadcasted_iota(jnp.int32, (H, D), 0))


def _prologue_kernel(mt_ref, g_ref, be_ref, wq_ref, bq_ref, a_ref):
    x = mt_ref[...]
    mu = jnp.mean(x, axis=1, keepdims=True)
    var = jnp.mean((x - mu) ** 2, axis=1, keepdims=True)
    qin = (x - mu) / jnp.sqrt(var + 1e-6) * g_ref[...] + be_ref[...]
    q = _dot(qin, wq_ref[...], ((1,), (1,))) + bq_ref[...]
    a_ref[...] = jnp.where(_headmask(), jnp.broadcast_to(q, (H, D)), 0.0)


def _gate_kernel(t_ref, a_ref, wk_ref, bk_ref, wv_ref, bv_ref,
                 wo_ref, bo_ref, gw_ref, h_ref,
                 idx_ref, w_ref, aux_ref,
                 sc_s, v_s, lg_s):
    b = pl.program_id(0)
    s = pl.program_id(1)
    nb = pl.num_programs(0)
    ns = pl.num_programs(1)

    for i in range(NMB):
        hb = h_ref[0, i * MBLK:(i + 1) * MBLK]
        kb = _dot(hb, wk_ref[...], ((1,), (1,))) + bk_ref[...]
        sc_s[:, pl.ds(s * SBLK + i * MBLK, MBLK)] = (
            _dot(a_ref[...], kb, ((1,), (1,))) * 0.125)
        v_s[pl.ds(s * SBLK + i * MBLK, MBLK), :] = (
            _dot(hb, wv_ref[...], ((1,), (1,))) + bv_ref[...])

    @pl.when(s == ns - 1)
    def _per_batch():
        sc = sc_s[...]
        mx = jnp.max(sc, axis=1, keepdims=True)
        den = jnp.sum(jnp.exp(sc - mx), axis=1, keepdims=True)
        acc = jnp.zeros((H, D), jnp.float32)
        nch = (ns * SBLK) // MBLK
        for j in range(nch):
            att = jnp.exp(sc_s[:, j * MBLK:(j + 1) * MBLK] - mx) / den
            acc = acc + _dot(att, v_s[j * MBLK:(j + 1) * MBLK, :],
                             ((1,), (0,)))
        ctx = jnp.sum(jnp.where(_headmask(), acc, 0.0), axis=0,
                      keepdims=True)
        out = _dot(ctx, wo_ref[...], ((1,), (1,))) + bo_ref[...]
        lg_s[pl.ds(b, 1), :] = _dot(out, gw_ref[...], ((1,), (1,)))

    @pl.when((b == nb - 1) & (s == ns - 1))
    def _routing():
        lg = lg_s[...]
        nbat = lg.shape[0]
        rowmax = jnp.max(lg, axis=1, keepdims=True)
        ex = jnp.exp(lg - rowmax)
        se = jnp.sum(ex, axis=1, keepdims=True)
        colio = jax.lax.broadcasted_iota(jnp.int32, (nbat, E), 1)
        work = ex / se
        idx_cols, w_cols = [], []
        tot = jnp.zeros((nbat, 1), jnp.float32)
        for _ in range(K):
            mx = jnp.max(work, axis=1, keepdims=True)
            cand = jnp.where(work == mx, colio, E)
            ix = jnp.min(cand, axis=1, keepdims=True)
            idx_cols.append(ix)
            w_cols.append(mx)
            tot = tot + mx
            work = jnp.where(colio == ix, -1.0, work)
        idx_ref[...] = jnp.concatenate(idx_cols, axis=1)
        w_ref[...] = jnp.concatenate(w_cols, axis=1) / (tot + 1e-20)
        t = t_ref[0]
        lse = jnp.log(se) + rowmax
        lt = jnp.sum(jnp.where(colio == t, lg, 0.0), axis=1, keepdims=True)
        aux_ref[...] = jnp.mean(lse - lt, keepdims=True)


def kernel(hidden_states, ln_gamma, ln_beta, moe_tokens, Wq, bq, Wk, bk,
           Wv, bv, Wo, bo, gate_weight, target_expert):
    B, S, d = hidden_states.shape
    ns = S // SBLK
    row = lambda v: v.reshape(1, d)
    t = jnp.asarray(target_expert, jnp.int32).reshape(1)

    A = pl.pallas_call(
        _prologue_kernel,
        out_shape=jax.ShapeDtypeStruct((H, d), jnp.float32),
    )(moe_tokens.reshape(1, d), row(ln_gamma), row(ln_beta), Wq, row(bq))

    const = lambda shape: pl.BlockSpec(shape, lambda b, s: (0,) * len(shape))

    idx, w, aux = pl.pallas_call(
        _gate_kernel,
        grid=(B, ns),
        in_specs=[
            pl.BlockSpec(memory_space=pltpu.SMEM),
            const((H, d)),
            const((d, d)), const((1, d)),
            const((d, d)), const((1, d)),
            const((d, d)), const((1, d)),
            const((E, d)),
            pl.BlockSpec((1, SBLK, d), lambda b, s: (b, s, 0)),
        ],
        out_specs=[const((B, K)), const((B, K)), const((1, 1))],
        out_shape=[
            jax.ShapeDtypeStruct((B, K), jnp.int32),
            jax.ShapeDtypeStruct((B, K), jnp.float32),
            jax.ShapeDtypeStruct((1, 1), jnp.float32),
        ],
        scratch_shapes=[
            pltpu.VMEM((H, S), jnp.float32),
            pltpu.VMEM((S, D), jnp.float32),
            pltpu.VMEM((B, E), jnp.float32),
        ],
        compiler_params=pltpu.CompilerParams(
            dimension_semantics=("arbitrary", "arbitrary")),
    )(t, A, Wk, row(bk), Wv, row(bv), Wo, row(bo), gate_weight,
      hidden_states)
    return idx, w, aux.reshape(())


# speedup vs baseline: 1.2142x; 1.0013x over previous
import jax
import jax.numpy as jnp
from jax.experimental import pallas as pl
from jax.experimental.pallas import tpu as pltpu

D = 768
H = 12
HD = 64
E = 64
K = 8
SBLK = 4096
MBLK = 1024
NMB = SBLK // MBLK


def _dot(a, b, dims):
    return jax.lax.dot_general(a, b, (dims, ((), ())),
                               preferred_element_type=jnp.float32)


def _headmask():
    return (jax.lax.broadcasted_iota(jnp.int32, (H, D), 1) // HD
            == jax.lax.broadcasted_iota(jnp.int32, (H, D), 0))


def _prologue_kernel(mt_ref, g_ref, be_ref, wq_ref, bq_ref, a_ref):
    x = mt_ref[...]
    mu = jnp.mean(x, axis=1, keepdims=True)
    var = jnp.mean((x - mu) ** 2, axis=1, keepdims=True)
    qin = (x - mu) / jnp.sqrt(var + 1e-6) * g_ref[...] + be_ref[...]
    q = _dot(qin, wq_ref[...], ((1,), (1,))) + bq_ref[...]
    a_ref[...] = jnp.where(_headmask(), jnp.broadcast_to(q, (H, D)), 0.0)


def _gate_kernel(t_ref, a_ref, wk_ref, bk_ref, wv_ref, bv_ref,
                 wo_ref, bo_ref, gw_ref, h_ref,
                 idx_ref, w_ref, aux_ref,
                 sc_s, v_s, lg_s):
    b = pl.program_id(0)
    s = pl.program_id(1)
    nb = pl.num_programs(0)
    ns = pl.num_programs(1)

    for i in range(NMB):
        hb = h_ref[0, i * MBLK:(i + 1) * MBLK]
        kb = _dot(hb, wk_ref[...], ((1,), (1,))) + bk_ref[...]
        sc_s[:, pl.ds(s * SBLK + i * MBLK, MBLK)] = (
            _dot(a_ref[...], kb, ((1,), (1,))) * 0.125)
        v_s[pl.ds(s * SBLK + i * MBLK, MBLK), :] = (
            _dot(hb, wv_ref[...], ((1,), (1,))) + bv_ref[...])

    @pl.when(s == ns - 1)
    def _per_batch():
        sc = sc_s[...]
        mx = jnp.max(sc, axis=1, keepdims=True)
        ex = jnp.exp(sc - mx)
        den = jnp.sum(ex, axis=1, keepdims=True)
        sc_s[...] = ex
        acc = jnp.zeros((H, D), jnp.float32)
        nch = (ns * SBLK) // MBLK
        for j in range(nch):
            att = sc_s[:, j * MBLK:(j + 1) * MBLK] / den
            acc = acc + _dot(att, v_s[j * MBLK:(j + 1) * MBLK, :],
                             ((1,), (0,)))
        ctx = jnp.sum(jnp.where(_headmask(), acc, 0.0), axis=0,
                      keepdims=True)
        out = _dot(ctx, wo_ref[...], ((1,), (1,))) + bo_ref[...]
        lg_s[pl.ds(b, 1), :] = _dot(out, gw_ref[...], ((1,), (1,)))

    @pl.when((b == nb - 1) & (s == ns - 1))
    def _routing():
        lg = lg_s[...]
        nbat = lg.shape[0]
        rowmax = jnp.max(lg, axis=1, keepdims=True)
        ex = jnp.exp(lg - rowmax)
        se = jnp.sum(ex, axis=1, keepdims=True)
        colio = jax.lax.broadcasted_iota(jnp.int32, (nbat, E), 1)
        work = ex / se
        idx_cols, w_cols = [], []
        tot = jnp.zeros((nbat, 1), jnp.float32)
        for _ in range(K):
            mx = jnp.max(work, axis=1, keepdims=True)
            cand = jnp.where(work == mx, colio, E)
            ix = jnp.min(cand, axis=1, keepdims=True)
            idx_cols.append(ix)
            w_cols.append(mx)
            tot = tot + mx
            work = jnp.where(colio == ix, -1.0, work)
        idx_ref[...] = jnp.concatenate(idx_cols, axis=1)
        w_ref[...] = jnp.concatenate(w_cols, axis=1) / (tot + 1e-20)
        t = t_ref[0]
        lse = jnp.log(se) + rowmax
        lt = jnp.sum(jnp.where(colio == t, lg, 0.0), axis=1, keepdims=True)
        aux_ref[...] = jnp.mean(lse - lt, keepdims=True)


def kernel(hidden_states, ln_gamma, ln_beta, moe_tokens, Wq, bq, Wk, bk,
           Wv, bv, Wo, bo, gate_weight, target_expert):
    B, S, d = hidden_states.shape
    ns = S // SBLK
    row = lambda v: v.reshape(1, d)
    t = jnp.asarray(target_expert, jnp.int32).reshape(1)

    A = pl.pallas_call(
        _prologue_kernel,
        out_shape=jax.ShapeDtypeStruct((H, d), jnp.float32),
    )(moe_tokens.reshape(1, d), row(ln_gamma), row(ln_beta), Wq, row(bq))

    const = lambda shape: pl.BlockSpec(shape, lambda b, s: (0,) * len(shape))

    idx, w, aux = pl.pallas_call(
        _gate_kernel,
        grid=(B, ns),
        in_specs=[
            pl.BlockSpec(memory_space=pltpu.SMEM),
            const((H, d)),
            const((d, d)), const((1, d)),
            const((d, d)), const((1, d)),
            const((d, d)), const((1, d)),
            const((E, d)),
            pl.BlockSpec((1, SBLK, d), lambda b, s: (b, s, 0)),
        ],
        out_specs=[const((B, K)), const((B, K)), const((1, 1))],
        out_shape=[
            jax.ShapeDtypeStruct((B, K), jnp.int32),
            jax.ShapeDtypeStruct((B, K), jnp.float32),
            jax.ShapeDtypeStruct((1, 1), jnp.float32),
        ],
        scratch_shapes=[
            pltpu.VMEM((H, S), jnp.float32),
            pltpu.VMEM((S, D), jnp.float32),
            pltpu.VMEM((B, E), jnp.float32),
        ],
        compiler_params=pltpu.CompilerParams(
            dimension_semantics=("arbitrary", "arbitrary")),
    )(t, A, Wk, row(bk), Wv, row(bv), Wo, row(bo), gate_weight,
      hidden_states)
    return idx, w, aux.reshape(())
